# baseline (speedup 1.0000x reference)
import functools

import jax
import jax.numpy as jnp
from jax import lax
from jax.experimental import pallas as pl
from jax.experimental.pallas import tpu as pltpu
from jax.experimental.pallas import tpu_sc as plsc

HIDDEN = 64
B, L = 4096, 200
N = B * L
NTOK = 1000000
TBLK = 2048
NTOKP = ((NTOK + TBLK - 1) // TBLK) * TBLK
NC, NS = 2, 16
NW = NC * NS
NPW = N // NW
G = 128
NG = NPW // G
NBUF = 2


def _tc_prep(tokT, posT, typT, turT, ptabT, ttab, utab):
    def body(tok_ref, p_ref, t_ref, u_ref, ptab_ref, ttab_ref, utab_ref,
             ftok_ref, fidx_ref, fused_ref):
        tok = tok_ref[...]
        slot = ((tok & 1023) << 1) | ((tok >> 10) & 1)
        ftok_ref[...] = (tok & ~2047) | slot
        fidx_ref[...] = p_ref[...] * 32 + t_ref[...] * 16 + u_ref[...]
        pos = ptab_ref[...].T
        pos2 = jnp.concatenate([pos, pos], axis=1)
        typ = ttab_ref[...]
        typ2 = jnp.concatenate([typ, typ], axis=1)
        tt3 = utab_ref[...].reshape(8, 2, 64)
        turnp = jnp.concatenate([tt3[:, 0, :], tt3[:, 1, :]], axis=1)
        fused_ref[...] = (pos2[:, None, None, :] + typ2[None, :, None, :]
                          + turnp[None, None, :, :])

    return pl.pallas_call(
        body,
        out_shape=[
            jax.ShapeDtypeStruct((L, B), jnp.int32),
            jax.ShapeDtypeStruct((L, B), jnp.int32),
            jax.ShapeDtypeStruct((512, 2, 8, 128), jnp.float32),
        ],
    )(tokT, posT, typT, turT, ptabT, ttab, utab)


def _tc_table_transpose(tabT):
    def body(x_ref, o_ref):
        y = x_ref[...].T
        o_ref[...] = jnp.concatenate([y[: TBLK // 2], y[TBLK // 2:]], axis=1)

    return pl.pallas_call(
        body,
        grid=(NTOKP // TBLK,),
        in_specs=[pl.BlockSpec((64, TBLK), lambda i: (0, i))],
        out_specs=pl.BlockSpec((TBLK // 2, 128), lambda i: (i, 0)),
        out_shape=jax.ShapeDtypeStruct((NTOKP // 2, 128), jnp.float32),
    )(tabT)


def _sc_embed(token_table, fused_table, tok_idx, fidx):
    mesh = plsc.VectorSubcoreMesh(core_axis_name="c", subcore_axis_name="s")

    @functools.partial(
        pl.kernel,
        out_type=jax.ShapeDtypeStruct((L, 8, B // G, 8, G), jnp.float32),
        mesh=mesh,
        scratch_types=[
            pltpu.VMEM((NG, G), jnp.int32),
            pltpu.VMEM((NG, G), jnp.int32),
            pltpu.VMEM((NBUF, G, HIDDEN), jnp.float32),
            pltpu.VMEM((NBUF, G, HIDDEN), jnp.float32),
            pltpu.VMEM((NBUF, HIDDEN, G), jnp.float32),
            pltpu.SemaphoreType.DMA,
            pltpu.SemaphoreType.DMA,
            pltpu.SemaphoreType.DMA,
            pltpu.SemaphoreType.DMA,
        ],
        compiler_params=pltpu.CompilerParams(use_tc_tiling_on_sc=False,
                                             needs_layout_passes=False),
    )
    def kern(tok_tab, fus_tab, tok_i, fus_i, out, idx_t, idx_f, rows_t,
             rows_f, tbuf, sg0, sg1, so0, so1):
        wid = lax.axis_index("s") * NC + lax.axis_index("c")
        base = wid * NPW
        pltpu.sync_copy(tok_i.at[wid], idx_t)
        pltpu.sync_copy(fus_i.at[wid], idx_f)
        sems = [sg0, sg1]
        sems_o = [so0, so1]
        col = lax.iota(jnp.int32, 16)

        def store_tiles(g, b):
            p = base + g * G
            l = p // B
            bb = (p % B) // G
            for hb in range(8):
                pltpu.make_async_copy(
                    tbuf.at[b, pl.ds(8 * hb, 8)], out.at[l, hb, bb],
                    sems_o[b]).start()

        def wait_tiles(g, b):
            p = base + g * G
            l = p // B
            bb = (p % B) // G
            for hb in range(8):
                pltpu.make_async_copy(
                    tbuf.at[b, pl.ds(8 * hb, 8)], out.at[l, hb, bb],
                    sems_o[b]).wait()

        def fire(g, b):
            pltpu.make_async_copy(
                tok_tab.at[idx_t.at[g]], rows_t.at[b], sems[b]).start()
            pltpu.make_async_copy(
                fus_tab.at[idx_f.at[g]], rows_f.at[b], sems[b]).start()

        def drain(g, b):
            pltpu.make_async_copy(
                tok_tab.at[idx_t.at[g]], rows_t.at[b], sems[b]).wait()
            pltpu.make_async_copy(
                fus_tab.at[idx_f.at[g]], rows_f.at[b], sems[b]).wait()

        for b in range(NBUF):
            fire(b, b)

        def outer(g0, carry):
            for b in range(NBUF):
                g = g0 * NBUF + b
                drain(g, b)

                @pl.when(g >= NBUF)
                def _():
                    wait_tiles(g - NBUF, b)

                def feat(h, c):
                    for tb in range(G // 16):
                        t16 = col + tb * 16
                        hv = col * 0 + h
                        v = (plsc.load_gather(rows_t.at[b], [t16, hv])
                             + plsc.load_gather(rows_f.at[b], [t16, hv]))
                        tbuf[b, h, pl.ds(tb * 16, 16)] = v
                    return c

                lax.fori_loop(0, HIDDEN, feat, carry)
                store_tiles(g, b)

                @pl.when(g + NBUF < NG)
                def _():
                    fire(g + NBUF, b)
            return carry

        lax.fori_loop(0, NG // NBUF, outer, 0)
        for b in range(NBUF):
            wait_tiles(NG - NBUF + b, b)

    return kern(token_table, fused_table, tok_idx, fidx)


def kernel(token_inp, pos_inp, type_inp, turn_inp, token_table, pos_table,
           type_table, turn_table):
    ftokP, fidxP, fusedP = _tc_prep(
        token_inp.astype(jnp.int32).T, pos_inp.T, type_inp.T, turn_inp.T,
        pos_table.T, type_table, turn_table)
    table_pairs = _tc_table_transpose(token_table.T)
    out5 = _sc_embed(
        table_pairs.reshape(NTOKP, HIDDEN),
        fusedP.reshape(512 * 2 * 16, HIDDEN),
        ftokP.reshape(NW, NG, G),
        fidxP.reshape(NW, NG, G),
    )
    outT = out5.transpose(0, 1, 3, 2, 4).reshape(L, HIDDEN, B)
    return outT.transpose(2, 0, 1)


# speedup vs baseline: 2.3160x; 2.3160x over previous
import functools

import jax
import jax.numpy as jnp
from jax import lax
from jax.experimental import pallas as pl
from jax.experimental.pallas import tpu as pltpu
from jax.experimental.pallas import tpu_sc as plsc

HIDDEN = 64
B, L = 4096, 200
N = B * L
NTOK = 1000000
TBLK = 2048
NTOKP = ((NTOK + TBLK - 1) // TBLK) * TBLK
NC, NS = 2, 16
NW = NC * NS
NPW = N // NW
G = 128
NG = NPW // G
NBUF = 2


def _tc_prep(tokT, posT, typT, turT, ptabT, ttab, utab):
    def body(tok_ref, p_ref, t_ref, u_ref, ptab_ref, ttab_ref, utab_ref,
             ftok_ref, fidx_ref, fused_ref):
        tok = tok_ref[...]
        slot = ((tok & 1023) << 1) | ((tok >> 10) & 1)
        ftok_ref[...] = (tok & ~2047) | slot
        fidx_ref[...] = p_ref[...] * 32 + t_ref[...] * 16 + u_ref[...]
        pos = ptab_ref[...].T
        pos2 = jnp.concatenate([pos, pos], axis=1)
        typ = ttab_ref[...]
        typ2 = jnp.concatenate([typ, typ], axis=1)
        tt3 = utab_ref[...].reshape(8, 2, 64)
        turnp = jnp.concatenate([tt3[:, 0, :], tt3[:, 1, :]], axis=1)
        fused_ref[...] = (pos2[:, None, None, :] + typ2[None, :, None, :]
                          + turnp[None, None, :, :])

    return pl.pallas_call(
        body,
        out_shape=[
            jax.ShapeDtypeStruct((L, B), jnp.int32),
            jax.ShapeDtypeStruct((L, B), jnp.int32),
            jax.ShapeDtypeStruct((512, 2, 8, 128), jnp.float32),
        ],
    )(tokT, posT, typT, turT, ptabT, ttab, utab)


def _tc_table_transpose(tabT):
    def body(x_ref, o_ref):
        y = x_ref[...].T
        o_ref[...] = jnp.concatenate([y[: TBLK // 2], y[TBLK // 2:]], axis=1)

    return pl.pallas_call(
        body,
        grid=(NTOKP // TBLK,),
        in_specs=[pl.BlockSpec((64, TBLK), lambda i: (0, i))],
        out_specs=pl.BlockSpec((TBLK // 2, 128), lambda i: (i, 0)),
        out_shape=jax.ShapeDtypeStruct((NTOKP // 2, 128), jnp.float32),
    )(tabT)


def _sc_embed(token_table, fused_table, tok_idx, fidx):
    mesh = plsc.VectorSubcoreMesh(core_axis_name="c", subcore_axis_name="s")

    @functools.partial(
        pl.kernel,
        out_type=jax.ShapeDtypeStruct((N, HIDDEN), jnp.float32),
        mesh=mesh,
        scratch_types=[
            pltpu.VMEM((NG, G), jnp.int32),
            pltpu.VMEM((NG, G), jnp.int32),
            pltpu.VMEM((NBUF, G, HIDDEN), jnp.float32),
            pltpu.VMEM((NBUF, G, HIDDEN), jnp.float32),
            pltpu.SemaphoreType.DMA,
            pltpu.SemaphoreType.DMA,
        ],
        compiler_params=pltpu.CompilerParams(use_tc_tiling_on_sc=False),
    )
    def kern(tok_tab, fus_tab, tok_i, fus_i, out, idx_t, idx_f, rows_t,
             rows_f, sem0, sem1):
        wid = lax.axis_index("s") * NC + lax.axis_index("c")
        base = wid * NPW
        pltpu.sync_copy(tok_i.at[wid], idx_t)
        pltpu.sync_copy(fus_i.at[wid], idx_f)
        sems = [sem0, sem1]

        def fire(g, b):
            pltpu.make_async_copy(
                tok_tab.at[idx_t.at[g]], rows_t.at[b], sems[b]).start()
            pltpu.make_async_copy(
                fus_tab.at[idx_f.at[g]], rows_f.at[b], sems[b]).start()

        def drain(g, b):
            pltpu.make_async_copy(
                tok_tab.at[idx_t.at[g]], rows_t.at[b], sems[b]).wait()
            pltpu.make_async_copy(
                fus_tab.at[idx_f.at[g]], rows_f.at[b], sems[b]).wait()

        for b in range(NBUF):
            fire(b, b)

        def outer(g0, carry):
            for b in range(NBUF):
                g = g0 * NBUF + b
                drain(g, b)

                def add_row(r, c):
                    for cc in range(HIDDEN // 16):
                        sl = (b, r, pl.ds(cc * 16, 16))
                        plsc.addupdate(rows_t.at[sl], rows_f[sl])
                    return c

                lax.fori_loop(0, G, add_row, carry)
                pltpu.sync_copy(rows_t.at[b],
                                out.at[pl.ds(base + g * G, G)])

                @pl.when(g + NBUF < NG)
                def _():
                    fire(g + NBUF, b)
            return carry

        lax.fori_loop(0, NG // NBUF, outer, 0)

    return kern(token_table, fused_table, tok_idx, fidx)


def kernel(token_inp, pos_inp, type_inp, turn_inp, token_table, pos_table,
           type_table, turn_table):
    ftokP, fidxP, fusedP = _tc_prep(
        token_inp.astype(jnp.int32).T, pos_inp.T, type_inp.T, turn_inp.T,
        pos_table.T, type_table, turn_table)
    table_pairs = _tc_table_transpose(token_table.T)
    out_lin = _sc_embed(
        table_pairs.reshape(NTOKP, HIDDEN),
        fusedP.reshape(512 * 2 * 16, HIDDEN),
        ftokP.reshape(NW, NG, G),
        fidxP.reshape(NW, NG, G),
    )
    return out_lin.reshape(L, B, HIDDEN).transpose(1, 0, 2)


# speedup vs baseline: 2.5849x; 1.1161x over previous
import functools

import jax
import jax.numpy as jnp
from jax import lax
from jax.experimental import pallas as pl
from jax.experimental.pallas import tpu as pltpu
from jax.experimental.pallas import tpu_sc as plsc

HIDDEN = 64
B, L = 4096, 200
N = B * L
NTOK = 1000000
TBLK = 4096
NTOKP = ((NTOK + TBLK - 1) // TBLK) * TBLK
NC, NS = 2, 16
NW = NC * NS
NPW = N // NW
G = 128
NG = NPW // G
NBUF = 2


def _tc_prep(tokT, posT, typT, turT, ptabT, ttab, utab):
    def body(tok_ref, p_ref, t_ref, u_ref, ptab_ref, ttab_ref, utab_ref,
             ftok_ref, fidx_ref, fused_ref):
        tok = tok_ref[...]
        slot = ((tok & 2047) << 1) | ((tok >> 11) & 1)
        ftok_ref[...] = (tok & ~4095) | slot
        fidx_ref[...] = p_ref[...] * 32 + t_ref[...] * 16 + u_ref[...]
        pos = ptab_ref[...].T
        pos2 = jnp.concatenate([pos, pos], axis=1)
        typ = ttab_ref[...]
        typ2 = jnp.concatenate([typ, typ], axis=1)
        tt3 = utab_ref[...].reshape(8, 2, 64)
        turnp = jnp.concatenate([tt3[:, 0, :], tt3[:, 1, :]], axis=1)
        fused_ref[...] = (pos2[:, None, None, :] + typ2[None, :, None, :]
                          + turnp[None, None, :, :])

    return pl.pallas_call(
        body,
        out_shape=[
            jax.ShapeDtypeStruct((L, B), jnp.int32),
            jax.ShapeDtypeStruct((L, B), jnp.int32),
            jax.ShapeDtypeStruct((512, 2, 8, 128), jnp.float32),
        ],
    )(tokT, posT, typT, turT, ptabT, ttab, utab)


def _tc_table_transpose(tabT):
    def body(x_ref, o_ref):
        y = x_ref[...].T
        o_ref[...] = jnp.concatenate([y[: TBLK // 2], y[TBLK // 2:]], axis=1)

    return pl.pallas_call(
        body,
        grid=(NTOKP // TBLK,),
        in_specs=[pl.BlockSpec((64, TBLK), lambda i: (0, i))],
        out_specs=pl.BlockSpec((TBLK // 2, 128), lambda i: (i, 0)),
        out_shape=jax.ShapeDtypeStruct((NTOKP // 2, 128), jnp.float32),
    )(tabT)


def _sc_embed(token_table, fused_table, tok_idx, fidx):
    mesh = plsc.VectorSubcoreMesh(core_axis_name="c", subcore_axis_name="s")

    @functools.partial(
        pl.kernel,
        out_type=jax.ShapeDtypeStruct((N, HIDDEN), jnp.float32),
        mesh=mesh,
        scratch_types=[
            pltpu.VMEM((NG, G), jnp.int32),
            pltpu.VMEM((NG, G), jnp.int32),
            pltpu.VMEM((NBUF, G, HIDDEN), jnp.float32),
            pltpu.VMEM((NBUF, G, HIDDEN), jnp.float32),
            pltpu.SemaphoreType.DMA,
            pltpu.SemaphoreType.DMA,
        ],
        compiler_params=pltpu.CompilerParams(use_tc_tiling_on_sc=False),
    )
    def kern(tok_tab, fus_tab, tok_i, fus_i, out, idx_t, idx_f, rows_t,
             rows_f, sem0, sem1):
        wid = lax.axis_index("s") * NC + lax.axis_index("c")
        base = wid * NPW
        pltpu.sync_copy(tok_i.at[wid], idx_t)
        pltpu.sync_copy(fus_i.at[wid], idx_f)
        sems = [sem0, sem1]

        def fire(g, b):
            pltpu.make_async_copy(
                tok_tab.at[idx_t.at[g]], rows_t.at[b], sems[b]).start()
            pltpu.make_async_copy(
                fus_tab.at[idx_f.at[g]], rows_f.at[b], sems[b]).start()

        def drain(g, b):
            pltpu.make_async_copy(
                tok_tab.at[idx_t.at[g]], rows_t.at[b], sems[b]).wait()
            pltpu.make_async_copy(
                fus_tab.at[idx_f.at[g]], rows_f.at[b], sems[b]).wait()

        for b in range(NBUF):
            fire(b, b)

        def outer(g0, carry):
            for b in range(NBUF):
                g = g0 * NBUF + b
                drain(g, b)

                def add_row(r, c):
                    for cc in range(HIDDEN // 16):
                        sl = (b, r, pl.ds(cc * 16, 16))
                        plsc.addupdate(rows_t.at[sl], rows_f[sl])
                    return c

                lax.fori_loop(0, G, add_row, carry)
                pltpu.sync_copy(rows_t.at[b],
                                out.at[pl.ds(base + g * G, G)])

                @pl.when(g + NBUF < NG)
                def _():
                    fire(g + NBUF, b)
            return carry

        lax.fori_loop(0, NG // NBUF, outer, 0)

    return kern(token_table, fused_table, tok_idx, fidx)


def kernel(token_inp, pos_inp, type_inp, turn_inp, token_table, pos_table,
           type_table, turn_table):
    ftokP, fidxP, fusedP = _tc_prep(
        token_inp.astype(jnp.int32).T, pos_inp.T, type_inp.T, turn_inp.T,
        pos_table.T, type_table, turn_table)
    table_pairs = _tc_table_transpose(token_table.T)
    out_lin = _sc_embed(
        table_pairs.reshape(NTOKP, HIDDEN),
        fusedP.reshape(512 * 2 * 16, HIDDEN),
        ftokP.reshape(NW, NG, G),
        fidxP.reshape(NW, NG, G),
    )
    return out_lin.reshape(L, B, HIDDEN).transpose(1, 0, 2)


# speedup vs baseline: 2.7512x; 1.0643x over previous
import functools

import jax
import jax.numpy as jnp
from jax import lax
from jax.experimental import pallas as pl
from jax.experimental.pallas import tpu as pltpu
from jax.experimental.pallas import tpu_sc as plsc

HIDDEN = 64
B, L = 4096, 200
N = B * L
NTOK = 1000000
TBLK = 8192
NTOKP = ((NTOK + TBLK - 1) // TBLK) * TBLK
NC, NS = 2, 16
NW = NC * NS
NPW = N // NW
G = 128
NG = NPW // G
NBUF = 2


def _tc_prep(tokT, posT, typT, turT, ptabT, ttab, utab):
    def body(tok_ref, p_ref, t_ref, u_ref, ptab_ref, ttab_ref, utab_ref,
             ftok_ref, fidx_ref, fused_ref):
        tok = tok_ref[...]
        slot = ((tok & 4095) << 1) | ((tok >> 12) & 1)
        ftok_ref[...] = (tok & ~8191) | slot
        fidx_ref[...] = p_ref[...] * 32 + t_ref[...] * 16 + u_ref[...]
        pos = ptab_ref[...].T
        pos2 = jnp.concatenate([pos, pos], axis=1)
        typ = ttab_ref[...]
        typ2 = jnp.concatenate([typ, typ], axis=1)
        tt3 = utab_ref[...].reshape(8, 2, 64)
        turnp = jnp.concatenate([tt3[:, 0, :], tt3[:, 1, :]], axis=1)
        fused_ref[...] = (pos2[:, None, None, :] + typ2[None, :, None, :]
                          + turnp[None, None, :, :])

    return pl.pallas_call(
        body,
        out_shape=[
            jax.ShapeDtypeStruct((L, B), jnp.int32),
            jax.ShapeDtypeStruct((L, B), jnp.int32),
            jax.ShapeDtypeStruct((512, 2, 8, 128), jnp.float32),
        ],
    )(tokT, posT, typT, turT, ptabT, ttab, utab)


def _tc_table_transpose(tabT):
    def body(x_ref, o_ref):
        y = x_ref[...].T
        o_ref[...] = jnp.concatenate([y[: TBLK // 2], y[TBLK // 2:]], axis=1)

    return pl.pallas_call(
        body,
        grid=(NTOKP // TBLK,),
        in_specs=[pl.BlockSpec((64, TBLK), lambda i: (0, i))],
        out_specs=pl.BlockSpec((TBLK // 2, 128), lambda i: (i, 0)),
        out_shape=jax.ShapeDtypeStruct((NTOKP // 2, 128), jnp.float32),
    )(tabT)


def _sc_embed(token_table, fused_table, tok_idx, fidx):
    mesh = plsc.VectorSubcoreMesh(core_axis_name="c", subcore_axis_name="s")

    @functools.partial(
        pl.kernel,
        out_type=jax.ShapeDtypeStruct((N, HIDDEN), jnp.float32),
        mesh=mesh,
        scratch_types=[
            pltpu.VMEM((NG, G), jnp.int32),
            pltpu.VMEM((NG, G), jnp.int32),
            pltpu.VMEM((NBUF, G, HIDDEN), jnp.float32),
            pltpu.VMEM((NBUF, G, HIDDEN), jnp.float32),
            pltpu.SemaphoreType.DMA,
            pltpu.SemaphoreType.DMA,
        ],
        compiler_params=pltpu.CompilerParams(use_tc_tiling_on_sc=False),
    )
    def kern(tok_tab, fus_tab, tok_i, fus_i, out, idx_t, idx_f, rows_t,
             rows_f, sem0, sem1):
        wid = lax.axis_index("s") * NC + lax.axis_index("c")
        base = wid * NPW
        pltpu.sync_copy(tok_i.at[wid], idx_t)
        pltpu.sync_copy(fus_i.at[wid], idx_f)
        sems = [sem0, sem1]

        def fire(g, b):
            pltpu.make_async_copy(
                tok_tab.at[idx_t.at[g]], rows_t.at[b], sems[b]).start()
            pltpu.make_async_copy(
                fus_tab.at[idx_f.at[g]], rows_f.at[b], sems[b]).start()

        def drain(g, b):
            pltpu.make_async_copy(
                tok_tab.at[idx_t.at[g]], rows_t.at[b], sems[b]).wait()
            pltpu.make_async_copy(
                fus_tab.at[idx_f.at[g]], rows_f.at[b], sems[b]).wait()

        for b in range(NBUF):
            fire(b, b)

        def outer(g0, carry):
            for b in range(NBUF):
                g = g0 * NBUF + b
                drain(g, b)

                def add_row(r, c):
                    for cc in range(HIDDEN // 16):
                        sl = (b, r, pl.ds(cc * 16, 16))
                        plsc.addupdate(rows_t.at[sl], rows_f[sl])
                    return c

                lax.fori_loop(0, G, add_row, carry)
                pltpu.sync_copy(rows_t.at[b],
                                out.at[pl.ds(base + g * G, G)])

                @pl.when(g + NBUF < NG)
                def _():
                    fire(g + NBUF, b)
            return carry

        lax.fori_loop(0, NG // NBUF, outer, 0)

    return kern(token_table, fused_table, tok_idx, fidx)


def kernel(token_inp, pos_inp, type_inp, turn_inp, token_table, pos_table,
           type_table, turn_table):
    ftokP, fidxP, fusedP = _tc_prep(
        token_inp.astype(jnp.int32).T, pos_inp.T, type_inp.T, turn_inp.T,
        pos_table.T, type_table, turn_table)
    table_pairs = _tc_table_transpose(token_table.T)
    out_lin = _sc_embed(
        table_pairs.reshape(NTOKP, HIDDEN),
        fusedP.reshape(512 * 2 * 16, HIDDEN),
        ftokP.reshape(NW, NG, G),
        fidxP.reshape(NW, NG, G),
    )
    return out_lin.reshape(L, B, HIDDEN).transpose(1, 0, 2)


# speedup vs baseline: 2.8431x; 1.0334x over previous
import functools

import jax
import jax.numpy as jnp
from jax import lax
from jax.experimental import pallas as pl
from jax.experimental.pallas import tpu as pltpu
from jax.experimental.pallas import tpu_sc as plsc

HIDDEN = 64
B, L = 4096, 200
N = B * L
NTOK = 1000000
TBLK = 16384
NTOKP = ((NTOK + TBLK - 1) // TBLK) * TBLK
NC, NS = 2, 16
NW = NC * NS
NPW = N // NW
G = 128
NG = NPW // G
NBUF = 2


def _tc_prep(tokT, posT, typT, turT, ptabT, ttab, utab):
    def body(tok_ref, p_ref, t_ref, u_ref, ptab_ref, ttab_ref, utab_ref,
             ftok_ref, fidx_ref, fused_ref):
        tok = tok_ref[...]
        slot = ((tok & 8191) << 1) | ((tok >> 13) & 1)
        ftok_ref[...] = (tok & ~16383) | slot
        fidx_ref[...] = p_ref[...] * 32 + t_ref[...] * 16 + u_ref[...]
        pos = ptab_ref[...].T
        pos2 = jnp.concatenate([pos, pos], axis=1)
        typ = ttab_ref[...]
        typ2 = jnp.concatenate([typ, typ], axis=1)
        tt3 = utab_ref[...].reshape(8, 2, 64)
        turnp = jnp.concatenate([tt3[:, 0, :], tt3[:, 1, :]], axis=1)
        fused_ref[...] = (pos2[:, None, None, :] + typ2[None, :, None, :]
                          + turnp[None, None, :, :])

    return pl.pallas_call(
        body,
        out_shape=[
            jax.ShapeDtypeStruct((L, B), jnp.int32),
            jax.ShapeDtypeStruct((L, B), jnp.int32),
            jax.ShapeDtypeStruct((512, 2, 8, 128), jnp.float32),
        ],
    )(tokT, posT, typT, turT, ptabT, ttab, utab)


def _tc_table_transpose(tabT):
    def body(x_ref, o_ref):
        y = x_ref[...].T
        o_ref[...] = jnp.concatenate([y[: TBLK // 2], y[TBLK // 2:]], axis=1)

    return pl.pallas_call(
        body,
        grid=(NTOKP // TBLK,),
        in_specs=[pl.BlockSpec((64, TBLK), lambda i: (0, i))],
        out_specs=pl.BlockSpec((TBLK // 2, 128), lambda i: (i, 0)),
        out_shape=jax.ShapeDtypeStruct((NTOKP // 2, 128), jnp.float32),
    )(tabT)


def _sc_embed(token_table, fused_table, tok_idx, fidx):
    mesh = plsc.VectorSubcoreMesh(core_axis_name="c", subcore_axis_name="s")

    @functools.partial(
        pl.kernel,
        out_type=jax.ShapeDtypeStruct((N, HIDDEN), jnp.float32),
        mesh=mesh,
        scratch_types=[
            pltpu.VMEM((NG, G), jnp.int32),
            pltpu.VMEM((NG, G), jnp.int32),
            pltpu.VMEM((NBUF, G, HIDDEN), jnp.float32),
            pltpu.VMEM((NBUF, G, HIDDEN), jnp.float32),
            pltpu.SemaphoreType.DMA,
            pltpu.SemaphoreType.DMA,
        ],
        compiler_params=pltpu.CompilerParams(use_tc_tiling_on_sc=False),
    )
    def kern(tok_tab, fus_tab, tok_i, fus_i, out, idx_t, idx_f, rows_t,
             rows_f, sem0, sem1):
        wid = lax.axis_index("s") * NC + lax.axis_index("c")
        base = wid * NPW
        pltpu.sync_copy(tok_i.at[wid], idx_t)
        pltpu.sync_copy(fus_i.at[wid], idx_f)
        sems = [sem0, sem1]

        def fire(g, b):
            pltpu.make_async_copy(
                tok_tab.at[idx_t.at[g]], rows_t.at[b], sems[b]).start()
            pltpu.make_async_copy(
                fus_tab.at[idx_f.at[g]], rows_f.at[b], sems[b]).start()

        def drain(g, b):
            pltpu.make_async_copy(
                tok_tab.at[idx_t.at[g]], rows_t.at[b], sems[b]).wait()
            pltpu.make_async_copy(
                fus_tab.at[idx_f.at[g]], rows_f.at[b], sems[b]).wait()

        for b in range(NBUF):
            fire(b, b)

        def outer(g0, carry):
            for b in range(NBUF):
                g = g0 * NBUF + b
                drain(g, b)

                def add_row(r, c):
                    for cc in range(HIDDEN // 16):
                        sl = (b, r, pl.ds(cc * 16, 16))
                        plsc.addupdate(rows_t.at[sl], rows_f[sl])
                    return c

                lax.fori_loop(0, G, add_row, carry)
                pltpu.sync_copy(rows_t.at[b],
                                out.at[pl.ds(base + g * G, G)])

                @pl.when(g + NBUF < NG)
                def _():
                    fire(g + NBUF, b)
            return carry

        lax.fori_loop(0, NG // NBUF, outer, 0)

    return kern(token_table, fused_table, tok_idx, fidx)


def kernel(token_inp, pos_inp, type_inp, turn_inp, token_table, pos_table,
           type_table, turn_table):
    ftokP, fidxP, fusedP = _tc_prep(
        token_inp.astype(jnp.int32).T, pos_inp.T, type_inp.T, turn_inp.T,
        pos_table.T, type_table, turn_table)
    table_pairs = _tc_table_transpose(token_table.T)
    out_lin = _sc_embed(
        table_pairs.reshape(NTOKP, HIDDEN),
        fusedP.reshape(512 * 2 * 16, HIDDEN),
        ftokP.reshape(NW, NG, G),
        fidxP.reshape(NW, NG, G),
    )
    return out_lin.reshape(L, B, HIDDEN).transpose(1, 0, 2)
